# baseline (speedup 1.0000x reference)
import jax
import jax.numpy as jnp
from jax.experimental import pallas as pl
from jax.experimental.pallas import tpu as pltpu

B, C, N, E = 4, 128, 2048, 2048
T = 2
NB = N // T
P0 = B * T
BN_EPS = 1e-5


def _body(vf_ref, inc_lo_ref, inc_hi_ref, ef_ref, rows_ref, par_ref,
          vout_ref, eout_ref,
          inc_cache, y_cache, a_acc, stats_ref):
    s = pl.program_id(0)
    E2 = E // 2

    @pl.when(s < P0)
    def _phase0():
        b = s // T
        t = s % T
        inc_lo = inc_lo_ref[0].astype(jnp.bfloat16)
        inc_hi = inc_hi_ref[0].astype(jnp.bfloat16)
        inc_cache[b, pl.ds(t * NB, NB), :E2] = inc_lo
        inc_cache[b, pl.ds(t * NB, NB), E2:] = inc_hi
        vf_t = vf_ref[0].astype(jnp.bfloat16)
        ap_lo = jnp.dot(vf_t, inc_lo, preferred_element_type=jnp.float32)
        ap_hi = jnp.dot(vf_t, inc_hi, preferred_element_type=jnp.float32)

        @pl.when(t == 0)
        def _first():
            a_acc[:, :E2] = ap_lo
            a_acc[:, E2:] = ap_hi

        @pl.when(t != 0)
        def _rest():
            a_acc[:, :E2] += ap_lo
            a_acc[:, E2:] += ap_hi

        @pl.when(t == T - 1)
        def _finish():
            ied = rows_ref[0, 0:1, :]
            a = (a_acc[...] * ied).astype(jnp.bfloat16)
            w1 = par_ref[:, 0:C].astype(jnp.bfloat16)
            w2 = par_ref[:, C:2 * C].astype(jnp.bfloat16)
            bcol = par_ref[:, 2 * C:2 * C + 1]
            ef = ef_ref[0].astype(jnp.bfloat16)
            y = (jnp.dot(w1, ef, preferred_element_type=jnp.float32)
                 + jnp.dot(w2, a, preferred_element_type=jnp.float32)
                 + bcol)
            y_cache[b] = y.astype(jnp.bfloat16)
            st = jnp.concatenate(
                [jnp.sum(y, axis=1, keepdims=True),
                 jnp.sum(y * y, axis=1, keepdims=True)], axis=1)

            @pl.when(b == 0)
            def _init():
                stats_ref[...] = st

            @pl.when(b != 0)
            def _acc():
                stats_ref[...] += st

    @pl.when(s >= P0)
    def _phase1():
        b = s - P0
        cnt = float(B * E)
        mean = stats_ref[:, 0:1] / cnt
        var = stats_ref[:, 1:2] / cnt - mean * mean
        scale = par_ref[:, 2 * C + 1:2 * C + 2] * jax.lax.rsqrt(var + BN_EPS)
        shift = par_ref[:, 2 * C + 2:2 * C + 3] - mean * scale
        z = y_cache[b].astype(jnp.float32) * scale + shift
        z = jnp.where(z >= 0, z, 0.2 * z)
        eout_ref[0] = z
        es = rows_ref[0, 1:2, :]
        zz = (z * es).astype(jnp.bfloat16)
        inc_b = inc_cache[b]
        v = jax.lax.dot_general(zz, inc_b, (((1,), (1,)), ((), ())),
                                preferred_element_type=jnp.float32)
        ivd = rows_ref[0, 2:3, :]
        vout_ref[0] = v * ivd


@jax.jit
def kernel(vertex_feat, edge_feat, edge_weight, incidence, inv_edge_degree,
           inv_vertex_degree, edge_scale, knn_k, conv_w, conv_b, bn_gamma,
           bn_beta):
    del edge_weight, knn_k
    par = jnp.concatenate(
        [conv_w, conv_b[:, None], bn_gamma[:, None], bn_beta[:, None]],
        axis=1)
    rows = jnp.concatenate(
        [inv_edge_degree[:, :, 0][:, None, :],
         edge_scale[:, :, 0][:, None, :],
         inv_vertex_degree[:, :, 0][:, None, :]], axis=1)

    def b0(s):
        return jnp.where(s < P0, s // T, 0)

    def b_any(s):
        return jnp.where(s < P0, s // T, s - P0)

    def b1(s):
        return jnp.where(s < P0, 0, s - P0)

    vout, eout = pl.pallas_call(
        _body,
        grid=(P0 + B,),
        in_specs=[
            pl.BlockSpec((1, C, NB),
                         lambda s: (b0(s), 0, jnp.where(s < P0, s % T, 0))),
            pl.BlockSpec((1, NB, E // 2),
                         lambda s: (b0(s), jnp.where(s < P0, s % T, 0), 0)),
            pl.BlockSpec((1, NB, E // 2),
                         lambda s: (b0(s), jnp.where(s < P0, s % T, 0), 1)),
            pl.BlockSpec((1, C, E), lambda s: (b0(s), 0, 0)),
            pl.BlockSpec((1, 3, E), lambda s: (b_any(s), 0, 0)),
            pl.BlockSpec((C, 2 * C + 3), lambda s: (0, 0)),
        ],
        out_specs=[
            pl.BlockSpec((1, C, N), lambda s: (b1(s), 0, 0)),
            pl.BlockSpec((1, C, E), lambda s: (b1(s), 0, 0)),
        ],
        out_shape=[
            jax.ShapeDtypeStruct((B, C, N), jnp.float32),
            jax.ShapeDtypeStruct((B, C, E), jnp.float32),
        ],
        scratch_shapes=[
            pltpu.VMEM((B, N, E), jnp.bfloat16),
            pltpu.VMEM((B, C, E), jnp.bfloat16),
            pltpu.VMEM((C, E), jnp.float32),
            pltpu.VMEM((C, 2), jnp.float32),
        ],
        compiler_params=pltpu.CompilerParams(
            vmem_limit_bytes=63 * 1024 * 1024),
    )(vertex_feat, incidence, incidence, edge_feat, rows, par)

    return (vout, eout)


# speedup vs baseline: 1.0210x; 1.0210x over previous
import jax
import jax.numpy as jnp
from jax.experimental import pallas as pl
from jax.experimental.pallas import tpu as pltpu

B, C, N, E = 4, 128, 2048, 2048
T = 2
NB = N // T
P0 = B * T
BN_EPS = 1e-5


def _body(vf_ref, inc_ref, ef_ref, rows_ref, par_ref,
          vout_ref, eout_ref,
          inc_cache, y_cache, a_acc, stats_ref):
    s = pl.program_id(0)

    @pl.when(s < P0)
    def _phase0():
        b = s // T
        t = s % T
        inc_bf = inc_ref[0].astype(jnp.bfloat16)
        inc_cache[b, pl.ds(t * NB, NB), :] = inc_bf
        vf_t = vf_ref[0].astype(jnp.bfloat16)
        ap = jnp.dot(vf_t, inc_bf, preferred_element_type=jnp.float32)

        @pl.when(t == 0)
        def _first():
            a_acc[...] = ap

        @pl.when(t != 0)
        def _rest():
            a_acc[...] += ap

        @pl.when(t == T - 1)
        def _finish():
            ied = rows_ref[0, 0:1, :]
            a = (a_acc[...] * ied).astype(jnp.bfloat16)
            w1 = par_ref[:, 0:C].astype(jnp.bfloat16)
            w2 = par_ref[:, C:2 * C].astype(jnp.bfloat16)
            bcol = par_ref[:, 2 * C:2 * C + 1]
            ef = ef_ref[0].astype(jnp.bfloat16)
            y = (jnp.dot(w1, ef, preferred_element_type=jnp.float32)
                 + jnp.dot(w2, a, preferred_element_type=jnp.float32)
                 + bcol)
            y_cache[b] = y
            st = jnp.concatenate(
                [jnp.sum(y, axis=1, keepdims=True),
                 jnp.sum(y * y, axis=1, keepdims=True)], axis=1)

            @pl.when(b == 0)
            def _init():
                stats_ref[...] = st

            @pl.when(b != 0)
            def _acc():
                stats_ref[...] += st

    @pl.when(s >= P0)
    def _phase1():
        b = s - P0
        cnt = float(B * E)
        mean = stats_ref[:, 0:1] / cnt
        var = stats_ref[:, 1:2] / cnt - mean * mean
        scale = par_ref[:, 2 * C + 1:2 * C + 2] * jax.lax.rsqrt(var + BN_EPS)
        shift = par_ref[:, 2 * C + 2:2 * C + 3] - mean * scale
        z = y_cache[b] * scale + shift
        z = jnp.where(z >= 0, z, 0.2 * z)
        eout_ref[0] = z
        es = rows_ref[0, 1:2, :]
        zz = (z * es).astype(jnp.bfloat16)
        inc_b = inc_cache[b]
        v = jax.lax.dot_general(zz, inc_b, (((1,), (1,)), ((), ())),
                                preferred_element_type=jnp.float32)
        ivd = rows_ref[0, 2:3, :]
        vout_ref[0] = v * ivd


@jax.jit
def kernel(vertex_feat, edge_feat, edge_weight, incidence, inv_edge_degree,
           inv_vertex_degree, edge_scale, knn_k, conv_w, conv_b, bn_gamma,
           bn_beta):
    del edge_weight, knn_k
    par = jnp.concatenate(
        [conv_w, conv_b[:, None], bn_gamma[:, None], bn_beta[:, None]],
        axis=1)
    rows = jnp.concatenate(
        [inv_edge_degree[:, :, 0][:, None, :],
         edge_scale[:, :, 0][:, None, :],
         inv_vertex_degree[:, :, 0][:, None, :]], axis=1)

    def b0(s):
        return jnp.where(s < P0, s // T, 0)

    def b_any(s):
        return jnp.where(s < P0, s // T, s - P0)

    def b1(s):
        return jnp.where(s < P0, 0, s - P0)

    vout, eout = pl.pallas_call(
        _body,
        grid=(P0 + B,),
        in_specs=[
            pl.BlockSpec((1, C, NB),
                         lambda s: (b0(s), 0, jnp.where(s < P0, s % T, 0))),
            pl.BlockSpec((1, NB, E),
                         lambda s: (b0(s), jnp.where(s < P0, s % T, 0), 0)),
            pl.BlockSpec((1, C, E), lambda s: (b0(s), 0, 0)),
            pl.BlockSpec((1, 3, E), lambda s: (b_any(s), 0, 0)),
            pl.BlockSpec((C, 2 * C + 3), lambda s: (0, 0)),
        ],
        out_specs=[
            pl.BlockSpec((1, C, N), lambda s: (b1(s), 0, 0)),
            pl.BlockSpec((1, C, E), lambda s: (b1(s), 0, 0)),
        ],
        out_shape=[
            jax.ShapeDtypeStruct((B, C, N), jnp.float32),
            jax.ShapeDtypeStruct((B, C, E), jnp.float32),
        ],
        scratch_shapes=[
            pltpu.VMEM((B, N, E), jnp.bfloat16),
            pltpu.VMEM((B, C, E), jnp.float32),
            pltpu.VMEM((C, E), jnp.float32),
            pltpu.VMEM((C, 2), jnp.float32),
        ],
        compiler_params=pltpu.CompilerParams(
            vmem_limit_bytes=63 * 1024 * 1024),
    )(vertex_feat, incidence, edge_feat, rows, par)

    return (vout, eout)
